# Initial kernel scaffold; baseline (speedup 1.0000x reference)
import functools

import jax
import jax.numpy as jnp
from jax import lax
from jax.experimental import pallas as pl
from jax.experimental.pallas import tpu as pltpu
from jax.experimental.pallas import tpu_sc as plsc

N_NODES = 10000
N_EDGES = 320000
D_IN = 128
D_HID = 64
D_OUT = 128

NC = 2
NS = 16
NW = NC * NS
EB = 125
NB = N_EDGES // (NW * EB)
ROWS_PT = N_NODES // NS
DEGW = 16

_SC_MESH = plsc.VectorSubcoreMesh(core_axis_name="c", subcore_axis_name="s")


@functools.partial(
    pl.kernel,
    out_type=jax.ShapeDtypeStruct((NC, N_NODES, DEGW), jnp.float32),
    mesh=_SC_MESH,
    scratch_types=[
        pltpu.VMEM((NB, EB), jnp.int32),
        pltpu.VMEM((EB, DEGW), jnp.float32),
        pltpu.MemoryRef((N_NODES, DEGW), jnp.float32, memory_space=pltpu.VMEM_SHARED),
    ],
)
def _deg_kernel(dst_hbm, ones_hbm, zeros_hbm, out_hbm, idx_d, ones_v, acc_sh):
    c = lax.axis_index("c")
    s = lax.axis_index("s")
    wid = c * NS + s
    pltpu.sync_copy(dst_hbm.at[pl.ds(wid * NB, NB)], idx_d)
    pltpu.sync_copy(ones_hbm, ones_v)
    pltpu.sync_copy(zeros_hbm, acc_sh.at[pl.ds(s * ROWS_PT, ROWS_PT)])
    plsc.subcore_barrier()

    def body(j, carry):
        pltpu.sync_copy(ones_v, acc_sh.at[idx_d.at[j]], add=True)
        return carry

    lax.fori_loop(0, NB, body, 0)
    plsc.subcore_barrier()
    pltpu.sync_copy(
        acc_sh.at[pl.ds(s * ROWS_PT, ROWS_PT)],
        out_hbm.at[c].at[pl.ds(s * ROWS_PT, ROWS_PT)],
    )


@functools.partial(
    pl.kernel,
    out_type=jax.ShapeDtypeStruct((NC, N_NODES, D_HID), jnp.float32),
    mesh=_SC_MESH,
    scratch_types=[
        pltpu.VMEM((NB, EB), jnp.int32),
        pltpu.VMEM((NB, EB), jnp.int32),
        pltpu.VMEM((EB, D_HID), jnp.float32),
        pltpu.MemoryRef((N_NODES, D_HID), jnp.float32, memory_space=pltpu.VMEM_SHARED),
        pltpu.SemaphoreType.DMA,
    ],
)
def _agg_kernel(y_hbm, src_hbm, dst_hbm, out_hbm, idx_s, idx_d, rows, acc_sh, sem):
    c = lax.axis_index("c")
    s = lax.axis_index("s")
    wid = c * NS + s
    pltpu.sync_copy(src_hbm.at[pl.ds(wid * NB, NB)], idx_s)
    pltpu.sync_copy(dst_hbm.at[pl.ds(wid * NB, NB)], idx_d)
    pltpu.sync_copy(
        y_hbm.at[pl.ds(s * ROWS_PT, ROWS_PT)],
        acc_sh.at[pl.ds(s * ROWS_PT, ROWS_PT)],
    )
    plsc.subcore_barrier()

    def body(j, carry):
        pltpu.async_copy(y_hbm.at[idx_s.at[j]], rows, sem).wait()
        pltpu.sync_copy(rows, acc_sh.at[idx_d.at[j]], add=True)
        return carry

    lax.fori_loop(0, NB, body, 0)
    plsc.subcore_barrier()
    pltpu.sync_copy(
        acc_sh.at[pl.ds(s * ROWS_PT, ROWS_PT)],
        out_hbm.at[c].at[pl.ds(s * ROWS_PT, ROWS_PT)],
    )


_RB = 1000


def _mm1_body(x_ref, w_ref, o_ref):
    o_ref[...] = jnp.dot(x_ref[...], w_ref[...], preferred_element_type=jnp.float32)


def _dinv_block(degp_ref):
    d = degp_ref[0] + degp_ref[1]
    return lax.rsqrt(1.0 + d[:, :1])


def _scale_body(degp_ref, xw_ref, o_ref):
    o_ref[...] = xw_ref[...] * _dinv_block(degp_ref)


def _hidden_body(degp_ref, zp_ref, y1_ref, b1_ref, o_ref):
    dinv = _dinv_block(degp_ref)
    z = zp_ref[0] + zp_ref[1] - y1_ref[...]
    h = jnp.maximum(z * dinv + b1_ref[...], 0.0)
    o_ref[...] = h * dinv


def _out_body(degp_ref, zp_ref, y2_ref, w2_ref, b2_ref, o_ref):
    dinv = _dinv_block(degp_ref)
    t = (zp_ref[0] + zp_ref[1] - y2_ref[...]) * dinv
    o_ref[...] = (
        jnp.dot(t, w2_ref[...], preferred_element_type=jnp.float32) + b2_ref[...]
    )


def _degp_spec():
    return pl.BlockSpec((NC, _RB, DEGW), lambda i: (0, i, 0))


def _zp_spec(w):
    return pl.BlockSpec((NC, _RB, w), lambda i: (0, i, 0))


def _row_spec(w):
    return pl.BlockSpec((_RB, w), lambda i: (i, 0))


def _full_spec(shape):
    return pl.BlockSpec(shape, lambda i: tuple(0 for _ in shape))


_GRID = (N_NODES // _RB,)


def kernel(x, edge_index, W1, b1, W2, b2):
    edge_index = edge_index.astype(jnp.int32)
    src = edge_index[0].reshape(NW * NB, EB)
    dst = edge_index[1].reshape(NW * NB, EB)
    ones_rows = jnp.ones((EB, DEGW), jnp.float32)
    zeros_stripe = jnp.zeros((ROWS_PT, DEGW), jnp.float32)

    degp = _deg_kernel(dst, ones_rows, zeros_stripe)

    xw1 = pl.pallas_call(
        _mm1_body,
        grid=_GRID,
        in_specs=[_row_spec(D_IN), _full_spec((D_IN, D_HID))],
        out_specs=_row_spec(D_HID),
        out_shape=jax.ShapeDtypeStruct((N_NODES, D_HID), jnp.float32),
    )(x, W1)

    y1 = pl.pallas_call(
        _scale_body,
        grid=_GRID,
        in_specs=[_degp_spec(), _row_spec(D_HID)],
        out_specs=_row_spec(D_HID),
        out_shape=jax.ShapeDtypeStruct((N_NODES, D_HID), jnp.float32),
    )(degp, xw1)

    zp1 = _agg_kernel(y1, src, dst)

    y2 = pl.pallas_call(
        _hidden_body,
        grid=_GRID,
        in_specs=[
            _degp_spec(),
            _zp_spec(D_HID),
            _row_spec(D_HID),
            _full_spec((1, D_HID)),
        ],
        out_specs=_row_spec(D_HID),
        out_shape=jax.ShapeDtypeStruct((N_NODES, D_HID), jnp.float32),
    )(degp, zp1, y1, b1.reshape(1, D_HID))

    zp2 = _agg_kernel(y2, src, dst)

    out = pl.pallas_call(
        _out_body,
        grid=_GRID,
        in_specs=[
            _degp_spec(),
            _zp_spec(D_HID),
            _row_spec(D_HID),
            _full_spec((D_HID, D_OUT)),
            _full_spec((1, D_OUT)),
        ],
        out_specs=_row_spec(D_OUT),
        out_shape=jax.ShapeDtypeStruct((N_NODES, D_OUT), jnp.float32),
    )(degp, zp2, y2, W2, b2.reshape(1, D_OUT))

    return out


# speedup vs baseline: 27.3483x; 27.3483x over previous
import functools

import jax
import jax.numpy as jnp
from jax import lax
from jax.experimental import pallas as pl
from jax.experimental.pallas import tpu as pltpu
from jax.experimental.pallas import tpu_sc as plsc

N_NODES = 10000
N_EDGES = 320000
D_IN = 128
D_HID = 64
D_OUT = 128

NC = 2
NS = 16
NW = NC * NS
EB = 125
NB = N_EDGES // (NW * EB)
N_P = 10240
ROWS_PT = N_P // NS
DEGW = 16

_SC_MESH = plsc.VectorSubcoreMesh(core_axis_name="c", subcore_axis_name="s")
_SC_PARAMS = pltpu.CompilerParams(use_tc_tiling_on_sc=False)


@functools.partial(
    pl.kernel,
    out_type=jax.ShapeDtypeStruct((NC, N_P, DEGW), jnp.float32),
    mesh=_SC_MESH,
    scratch_types=[
        pltpu.VMEM((NB, EB), jnp.int32),
        pltpu.VMEM((EB, DEGW), jnp.float32),
        pltpu.VMEM_SHARED((N_P, DEGW), jnp.float32),
    ],
    compiler_params=_SC_PARAMS,
)
def _deg_kernel(dst_hbm, ones_hbm, zeros_hbm, out_hbm, idx_d, ones_v, acc_sh):
    c = lax.axis_index("c")
    s = lax.axis_index("s")
    wid = c * NS + s
    pltpu.sync_copy(dst_hbm.at[pl.ds(wid * NB, NB)], idx_d)
    pltpu.sync_copy(ones_hbm, ones_v)
    pltpu.sync_copy(zeros_hbm, acc_sh.at[pl.ds(s * ROWS_PT, ROWS_PT)])
    plsc.subcore_barrier()

    def body(j, carry):
        pltpu.sync_copy(ones_v, acc_sh.at[idx_d.at[j]], add=True)
        return carry

    lax.fori_loop(0, NB, body, 0)
    plsc.subcore_barrier()
    pltpu.sync_copy(
        acc_sh.at[pl.ds(s * ROWS_PT, ROWS_PT)],
        out_hbm.at[c].at[pl.ds(s * ROWS_PT, ROWS_PT)],
    )


@functools.partial(
    pl.kernel,
    out_type=jax.ShapeDtypeStruct((NC, N_P, D_HID), jnp.float32),
    mesh=_SC_MESH,
    scratch_types=[
        pltpu.VMEM((NB, EB), jnp.int32),
        pltpu.VMEM((NB, EB), jnp.int32),
        pltpu.VMEM((EB, D_HID), jnp.float32),
        pltpu.VMEM_SHARED((N_P, D_HID), jnp.float32),
        pltpu.SemaphoreType.DMA,
    ],
    compiler_params=_SC_PARAMS,
)
def _agg_kernel(y_hbm, src_hbm, dst_hbm, out_hbm, idx_s, idx_d, rows, acc_sh, sem):
    c = lax.axis_index("c")
    s = lax.axis_index("s")
    wid = c * NS + s
    pltpu.sync_copy(src_hbm.at[pl.ds(wid * NB, NB)], idx_s)
    pltpu.sync_copy(dst_hbm.at[pl.ds(wid * NB, NB)], idx_d)
    pltpu.sync_copy(
        y_hbm.at[pl.ds(s * ROWS_PT, ROWS_PT)],
        acc_sh.at[pl.ds(s * ROWS_PT, ROWS_PT)],
    )
    plsc.subcore_barrier()

    def body(j, carry):
        pltpu.async_copy(y_hbm.at[idx_s.at[j]], rows, sem).wait()
        pltpu.sync_copy(rows, acc_sh.at[idx_d.at[j]], add=True)
        return carry

    lax.fori_loop(0, NB, body, 0)
    plsc.subcore_barrier()
    pltpu.sync_copy(
        acc_sh.at[pl.ds(s * ROWS_PT, ROWS_PT)],
        out_hbm.at[c].at[pl.ds(s * ROWS_PT, ROWS_PT)],
    )


_RB = 1024


def _mm1_body(x_ref, w_ref, o_ref):
    o_ref[...] = jnp.dot(x_ref[...], w_ref[...], preferred_element_type=jnp.float32)


def _dinv_block(degp_ref):
    d = degp_ref[0] + degp_ref[1]
    return lax.rsqrt(1.0 + d[:, :1])


def _scale_body(degp_ref, xw_ref, o_ref):
    o_ref[...] = xw_ref[...] * _dinv_block(degp_ref)


def _hidden_body(degp_ref, zp_ref, y1_ref, b1_ref, o_ref):
    dinv = _dinv_block(degp_ref)
    z = zp_ref[0] + zp_ref[1] - y1_ref[...]
    h = jnp.maximum(z * dinv + b1_ref[...], 0.0)
    o_ref[...] = h * dinv


def _out_body(degp_ref, zp_ref, y2_ref, w2_ref, b2_ref, o_ref):
    dinv = _dinv_block(degp_ref)
    t = (zp_ref[0] + zp_ref[1] - y2_ref[...]) * dinv
    o_ref[...] = (
        jnp.dot(t, w2_ref[...], preferred_element_type=jnp.float32) + b2_ref[...]
    )


def _degp_spec():
    return pl.BlockSpec((NC, _RB, DEGW), lambda i: (0, i, 0))


def _zp_spec(w):
    return pl.BlockSpec((NC, _RB, w), lambda i: (0, i, 0))


def _row_spec(w):
    return pl.BlockSpec((_RB, w), lambda i: (i, 0))


def _full_spec(shape):
    return pl.BlockSpec(shape, lambda i: tuple(0 for _ in shape))


_GRID = (N_P // _RB,)


def kernel(x, edge_index, W1, b1, W2, b2):
    edge_index = edge_index.astype(jnp.int32)
    src = edge_index[0].reshape(NW * NB, EB)
    dst = edge_index[1].reshape(NW * NB, EB)
    ones_rows = jnp.ones((EB, DEGW), jnp.float32)
    zeros_stripe = jnp.zeros((ROWS_PT, DEGW), jnp.float32)
    xp = jnp.pad(x, ((0, N_P - N_NODES), (0, 0)))

    degp = _deg_kernel(dst, ones_rows, zeros_stripe)

    xw1 = pl.pallas_call(
        _mm1_body,
        grid=_GRID,
        in_specs=[_row_spec(D_IN), _full_spec((D_IN, D_HID))],
        out_specs=_row_spec(D_HID),
        out_shape=jax.ShapeDtypeStruct((N_P, D_HID), jnp.float32),
    )(xp, W1)

    y1 = pl.pallas_call(
        _scale_body,
        grid=_GRID,
        in_specs=[_degp_spec(), _row_spec(D_HID)],
        out_specs=_row_spec(D_HID),
        out_shape=jax.ShapeDtypeStruct((N_P, D_HID), jnp.float32),
    )(degp, xw1)

    zp1 = _agg_kernel(y1, src, dst)

    y2 = pl.pallas_call(
        _hidden_body,
        grid=_GRID,
        in_specs=[
            _degp_spec(),
            _zp_spec(D_HID),
            _row_spec(D_HID),
            _full_spec((1, D_HID)),
        ],
        out_specs=_row_spec(D_HID),
        out_shape=jax.ShapeDtypeStruct((N_P, D_HID), jnp.float32),
    )(degp, zp1, y1, b1.reshape(1, D_HID))

    zp2 = _agg_kernel(y2, src, dst)

    out = pl.pallas_call(
        _out_body,
        grid=_GRID,
        in_specs=[
            _degp_spec(),
            _zp_spec(D_HID),
            _row_spec(D_HID),
            _full_spec((D_HID, D_OUT)),
            _full_spec((1, D_OUT)),
        ],
        out_specs=_row_spec(D_OUT),
        out_shape=jax.ShapeDtypeStruct((N_P, D_OUT), jnp.float32),
    )(degp, zp2, y2, W2, b2.reshape(1, D_OUT))

    return out[:N_NODES]


# speedup vs baseline: 32.1279x; 1.1748x over previous
import functools

import jax
import jax.numpy as jnp
from jax import lax
from jax.experimental import pallas as pl
from jax.experimental.pallas import tpu as pltpu
from jax.experimental.pallas import tpu_sc as plsc

N_NODES = 10000
N_EDGES = 320000
D_IN = 128
D_HID = 64
D_OUT = 128

NC = 2
NS = 16
NW = NC * NS
EB = 125
NB = N_EDGES // (NW * EB)
N_P = 10240
ROWS_PT = N_P // NS
DEGW = 16

_SC_MESH = plsc.VectorSubcoreMesh(core_axis_name="c", subcore_axis_name="s")
_SC_PARAMS = pltpu.CompilerParams(use_tc_tiling_on_sc=False)


@functools.partial(
    pl.kernel,
    out_type=jax.ShapeDtypeStruct((NC, N_P, DEGW), jnp.float32),
    mesh=_SC_MESH,
    scratch_types=[
        pltpu.VMEM((NB, EB), jnp.int32),
        pltpu.VMEM((EB, DEGW), jnp.float32),
        pltpu.VMEM_SHARED((N_P, DEGW), jnp.float32),
    ],
    compiler_params=_SC_PARAMS,
)
def _deg_kernel(dst_hbm, ones_hbm, zeros_hbm, out_hbm, idx_d, ones_v, acc_sh):
    c = lax.axis_index("c")
    s = lax.axis_index("s")
    wid = c * NS + s
    pltpu.sync_copy(dst_hbm.at[pl.ds(wid * NB, NB)], idx_d)
    pltpu.sync_copy(ones_hbm, ones_v)
    pltpu.sync_copy(zeros_hbm, acc_sh.at[pl.ds(s * ROWS_PT, ROWS_PT)])
    plsc.subcore_barrier()

    def body(j, carry):
        pltpu.sync_copy(ones_v, acc_sh.at[idx_d.at[j]], add=True)
        return carry

    lax.fori_loop(0, NB, body, 0)
    plsc.subcore_barrier()
    pltpu.sync_copy(
        acc_sh.at[pl.ds(s * ROWS_PT, ROWS_PT)],
        out_hbm.at[c].at[pl.ds(s * ROWS_PT, ROWS_PT)],
    )


@functools.partial(
    pl.kernel,
    out_type=jax.ShapeDtypeStruct((NC, N_P, D_HID), jnp.float32),
    mesh=_SC_MESH,
    scratch_types=[
        pltpu.VMEM((NB, EB), jnp.int32),
        pltpu.VMEM((NB, EB), jnp.int32),
        pltpu.VMEM((EB, D_HID), jnp.float32),
        pltpu.VMEM((EB, D_HID), jnp.float32),
        pltpu.VMEM_SHARED((N_P, D_HID), jnp.float32),
        pltpu.SemaphoreType.DMA,
        pltpu.SemaphoreType.DMA,
    ],
    compiler_params=_SC_PARAMS,
)
def _agg_kernel(y_hbm, src_hbm, dst_hbm, out_hbm, idx_s, idx_d, rows0, rows1,
                acc_sh, sem0, sem1):
    c = lax.axis_index("c")
    s = lax.axis_index("s")
    wid = c * NS + s
    pltpu.sync_copy(src_hbm.at[pl.ds(wid * NB, NB)], idx_s)
    pltpu.sync_copy(dst_hbm.at[pl.ds(wid * NB, NB)], idx_d)
    pltpu.sync_copy(
        y_hbm.at[pl.ds(s * ROWS_PT, ROWS_PT)],
        acc_sh.at[pl.ds(s * ROWS_PT, ROWS_PT)],
    )
    plsc.subcore_barrier()

    pltpu.async_copy(y_hbm.at[idx_s.at[0]], rows0, sem0)

    def body(i, carry):
        j0 = 2 * i
        j1 = j0 + 1
        pltpu.make_async_copy(y_hbm.at[idx_s.at[j0]], rows0, sem0).wait()
        pltpu.async_copy(y_hbm.at[idx_s.at[j1]], rows1, sem1)
        pltpu.sync_copy(rows0, acc_sh.at[idx_d.at[j0]], add=True)
        pltpu.make_async_copy(y_hbm.at[idx_s.at[j1]], rows1, sem1).wait()

        @pl.when(i < NB // 2 - 1)
        def _():
            pltpu.async_copy(y_hbm.at[idx_s.at[j0 + 2]], rows0, sem0)

        pltpu.sync_copy(rows1, acc_sh.at[idx_d.at[j1]], add=True)
        return carry

    lax.fori_loop(0, NB // 2, body, 0)
    plsc.subcore_barrier()
    pltpu.sync_copy(
        acc_sh.at[pl.ds(s * ROWS_PT, ROWS_PT)],
        out_hbm.at[c].at[pl.ds(s * ROWS_PT, ROWS_PT)],
    )


_RB = 1024


def _dinv_block(degp_ref):
    d = degp_ref[0] + degp_ref[1]
    return lax.rsqrt(1.0 + d[:, :1])


def _mm1_body(degp_ref, x_ref, w_ref, o_ref):
    xw = jnp.dot(x_ref[...], w_ref[...], preferred_element_type=jnp.float32)
    o_ref[...] = xw * _dinv_block(degp_ref)


def _hidden_body(degp_ref, zp_ref, y1_ref, b1_ref, o_ref):
    dinv = _dinv_block(degp_ref)
    z = zp_ref[0] + zp_ref[1] - y1_ref[...]
    h = jnp.maximum(z * dinv + b1_ref[...], 0.0)
    o_ref[...] = h * dinv


def _out_body(degp_ref, zp_ref, y2_ref, w2_ref, b2_ref, o_ref):
    dinv = _dinv_block(degp_ref)
    t = (zp_ref[0] + zp_ref[1] - y2_ref[...]) * dinv
    o_ref[...] = (
        jnp.dot(t, w2_ref[...], preferred_element_type=jnp.float32) + b2_ref[...]
    )


def _degp_spec():
    return pl.BlockSpec((NC, _RB, DEGW), lambda i: (0, i, 0))


def _zp_spec(w):
    return pl.BlockSpec((NC, _RB, w), lambda i: (0, i, 0))


def _row_spec(w):
    return pl.BlockSpec((_RB, w), lambda i: (i, 0))


def _full_spec(shape):
    return pl.BlockSpec(shape, lambda i: tuple(0 for _ in shape))


_GRID = (N_P // _RB,)


def kernel(x, edge_index, W1, b1, W2, b2):
    edge_index = edge_index.astype(jnp.int32)
    src = edge_index[0].reshape(NW * NB, EB)
    dst = edge_index[1].reshape(NW * NB, EB)
    ones_rows = jnp.ones((EB, DEGW), jnp.float32)
    zeros_stripe = jnp.zeros((ROWS_PT, DEGW), jnp.float32)
    xp = jnp.pad(x, ((0, N_P - N_NODES), (0, 0)))

    degp = _deg_kernel(dst, ones_rows, zeros_stripe)

    y1 = pl.pallas_call(
        _mm1_body,
        grid=_GRID,
        in_specs=[_degp_spec(), _row_spec(D_IN), _full_spec((D_IN, D_HID))],
        out_specs=_row_spec(D_HID),
        out_shape=jax.ShapeDtypeStruct((N_P, D_HID), jnp.float32),
    )(degp, xp, W1)

    zp1 = _agg_kernel(y1, src, dst)

    y2 = pl.pallas_call(
        _hidden_body,
        grid=_GRID,
        in_specs=[
            _degp_spec(),
            _zp_spec(D_HID),
            _row_spec(D_HID),
            _full_spec((1, D_HID)),
        ],
        out_specs=_row_spec(D_HID),
        out_shape=jax.ShapeDtypeStruct((N_P, D_HID), jnp.float32),
    )(degp, zp1, y1, b1.reshape(1, D_HID))

    zp2 = _agg_kernel(y2, src, dst)

    out = pl.pallas_call(
        _out_body,
        grid=_GRID,
        in_specs=[
            _degp_spec(),
            _zp_spec(D_HID),
            _row_spec(D_HID),
            _full_spec((D_HID, D_OUT)),
            _full_spec((1, D_OUT)),
        ],
        out_specs=_row_spec(D_OUT),
        out_shape=jax.ShapeDtypeStruct((N_P, D_OUT), jnp.float32),
    )(degp, zp2, y2, W2, b2.reshape(1, D_OUT))

    return out[:N_NODES]


# speedup vs baseline: 41.8087x; 1.3013x over previous
import functools

import jax
import jax.numpy as jnp
from jax import lax
from jax.experimental import pallas as pl
from jax.experimental.pallas import tpu as pltpu
from jax.experimental.pallas import tpu_sc as plsc

N_NODES = 10000
N_EDGES = 320000
D_IN = 128
D_HID = 64
D_OUT = 128

NC = 2
NS = 16
NW = NC * NS
EB = 128
NB = 80
E_P = NW * NB * EB
N_P = 10240
ROWS_PT = N_P // NS
DEGW = 8

_SC_MESH = plsc.VectorSubcoreMesh(core_axis_name="c", subcore_axis_name="s")
_SC_PARAMS = pltpu.CompilerParams(use_tc_tiling_on_sc=False)


@functools.partial(
    pl.kernel,
    out_type=jax.ShapeDtypeStruct((NC, N_P, DEGW), jnp.float32),
    mesh=_SC_MESH,
    scratch_types=[
        pltpu.VMEM((NB, EB), jnp.int32),
        pltpu.VMEM((EB, DEGW), jnp.float32),
        pltpu.VMEM_SHARED((N_P, DEGW), jnp.float32),
        pltpu.SemaphoreType.DMA,
        pltpu.SemaphoreType.DMA,
        pltpu.SemaphoreType.DMA,
        pltpu.SemaphoreType.DMA,
    ],
    compiler_params=_SC_PARAMS,
)
def _deg_kernel(er_hbm, ones_hbm, zeros_hbm, out_hbm, idx_d, ones_v, acc_sh,
                s0, s1, s2, s3):
    c = lax.axis_index("c")
    s = lax.axis_index("s")
    wid = c * NS + s
    pltpu.sync_copy(er_hbm.at[1].at[pl.ds(wid * NB, NB)], idx_d)
    pltpu.sync_copy(ones_hbm, ones_v)
    pltpu.sync_copy(zeros_hbm, acc_sh.at[pl.ds(s * ROWS_PT, ROWS_PT)])
    plsc.subcore_barrier()

    sems = (s0, s1, s2, s3)

    def body(i, carry):
        for b in range(4):
            pltpu.async_copy(
                ones_v, acc_sh.at[idx_d.at[4 * i + b]], sems[b], add=True
            )
        for b in range(4):
            pltpu.make_async_copy(
                ones_v, acc_sh.at[idx_d.at[4 * i + b]], sems[b]
            ).wait()
        return carry

    lax.fori_loop(0, NB // 4, body, 0)
    plsc.subcore_barrier()
    pltpu.sync_copy(
        acc_sh.at[pl.ds(s * ROWS_PT, ROWS_PT)],
        out_hbm.at[c].at[pl.ds(s * ROWS_PT, ROWS_PT)],
    )


@functools.partial(
    pl.kernel,
    out_type=jax.ShapeDtypeStruct((NC, N_P, D_HID), jnp.float32),
    mesh=_SC_MESH,
    scratch_types=[
        pltpu.VMEM((NB, EB), jnp.int32),
        pltpu.VMEM((NB, EB), jnp.int32),
        pltpu.VMEM((EB, D_HID), jnp.float32),
        pltpu.VMEM((EB, D_HID), jnp.float32),
        pltpu.VMEM((EB, D_HID), jnp.float32),
        pltpu.VMEM((EB, D_HID), jnp.float32),
        pltpu.VMEM_SHARED((N_P, D_HID), jnp.float32),
        pltpu.SemaphoreType.DMA,
        pltpu.SemaphoreType.DMA,
        pltpu.SemaphoreType.DMA,
        pltpu.SemaphoreType.DMA,
        pltpu.SemaphoreType.DMA,
        pltpu.SemaphoreType.DMA,
        pltpu.SemaphoreType.DMA,
        pltpu.SemaphoreType.DMA,
    ],
    compiler_params=_SC_PARAMS,
)
def _agg_kernel(y_hbm, er_hbm, out_hbm, idx_s, idx_d, r0, r1, r2, r3, acc_sh,
                g0, g1, g2, g3, t0, t1, t2, t3):
    c = lax.axis_index("c")
    s = lax.axis_index("s")
    wid = c * NS + s
    rows = (r0, r1, r2, r3)
    gsem = (g0, g1, g2, g3)
    ssem = (t0, t1, t2, t3)
    pltpu.sync_copy(er_hbm.at[0].at[pl.ds(wid * NB, NB)], idx_s)
    pltpu.sync_copy(er_hbm.at[1].at[pl.ds(wid * NB, NB)], idx_d)
    pltpu.sync_copy(
        y_hbm.at[pl.ds(s * ROWS_PT, ROWS_PT)],
        acc_sh.at[pl.ds(s * ROWS_PT, ROWS_PT)],
    )
    plsc.subcore_barrier()

    pltpu.async_copy(y_hbm.at[idx_s.at[0]], rows[0], gsem[0])
    pltpu.async_copy(y_hbm.at[idx_s.at[1]], rows[1], gsem[1])

    NI = NB // 4

    def body(i, carry):
        for b in range(4):
            j = 4 * i + b
            b2 = (b + 2) % 4
            pltpu.make_async_copy(y_hbm.at[idx_s.at[j]], rows[b], gsem[b]).wait()
            pltpu.async_copy(rows[b], acc_sh.at[idx_d.at[j]], ssem[b], add=True)
            if b < 2:
                @pl.when(i > 0)
                def _():
                    pltpu.make_async_copy(
                        rows[b2], acc_sh.at[idx_d.at[j - 2]], ssem[b2]
                    ).wait()

                pltpu.async_copy(y_hbm.at[idx_s.at[j + 2]], rows[b2], gsem[b2])
            else:
                @pl.when(i < NI - 1)
                def _():
                    pltpu.make_async_copy(
                        rows[b2], acc_sh.at[idx_d.at[j - 2]], ssem[b2]
                    ).wait()
                    pltpu.async_copy(y_hbm.at[idx_s.at[j + 2]], rows[b2], gsem[b2])
        return carry

    lax.fori_loop(0, NI, body, 0)
    for b in range(4):
        j = NB - 4 + b
        pltpu.make_async_copy(rows[b], acc_sh.at[idx_d.at[j]], ssem[b]).wait()
    plsc.subcore_barrier()
    pltpu.sync_copy(
        acc_sh.at[pl.ds(s * ROWS_PT, ROWS_PT)],
        out_hbm.at[c].at[pl.ds(s * ROWS_PT, ROWS_PT)],
    )


_RB = 1024


def _dinv_block(degp_ref):
    d = degp_ref[0] + degp_ref[1]
    return lax.rsqrt(1.0 + d[:, :1])


def _mm1_body(degp_ref, x_ref, w_ref, o_ref):
    xw = jnp.dot(x_ref[...], w_ref[...], preferred_element_type=jnp.float32)
    o_ref[...] = xw * _dinv_block(degp_ref)


def _hidden_body(degp_ref, zp_ref, y1_ref, b1_ref, o_ref):
    dinv = _dinv_block(degp_ref)
    z = zp_ref[0] + zp_ref[1] - y1_ref[...]
    h = jnp.maximum(z * dinv + b1_ref[...], 0.0)
    o_ref[...] = h * dinv


def _out_body(degp_ref, zp_ref, y2_ref, w2_ref, b2_ref, o_ref):
    dinv = _dinv_block(degp_ref)
    t = (zp_ref[0] + zp_ref[1] - y2_ref[...]) * dinv
    o_ref[...] = (
        jnp.dot(t, w2_ref[...], preferred_element_type=jnp.float32) + b2_ref[...]
    )


def _degp_spec():
    return pl.BlockSpec((NC, _RB, DEGW), lambda i: (0, i, 0))


def _zp_spec(w):
    return pl.BlockSpec((NC, _RB, w), lambda i: (0, i, 0))


def _row_spec(w):
    return pl.BlockSpec((_RB, w), lambda i: (i, 0))


def _full_spec(shape):
    return pl.BlockSpec(shape, lambda i: tuple(0 for _ in shape))


_GRID = (N_P // _RB,)


def kernel(x, edge_index, W1, b1, W2, b2):
    pad_e = E_P - N_EDGES
    pad_rows = N_NODES + (jnp.arange(pad_e, dtype=jnp.int32) % (N_P - N_NODES))
    er = jnp.concatenate(
        [edge_index.astype(jnp.int32), jnp.stack([pad_rows, pad_rows])], axis=1
    ).reshape(2, NW * NB, EB)
    ones_rows = jnp.ones((EB, DEGW), jnp.float32)
    zeros_stripe = jnp.zeros((ROWS_PT, DEGW), jnp.float32)

    degp = _deg_kernel(er, ones_rows, zeros_stripe)

    y1 = pl.pallas_call(
        _mm1_body,
        grid=_GRID,
        in_specs=[_degp_spec(), _row_spec(D_IN), _full_spec((D_IN, D_HID))],
        out_specs=_row_spec(D_HID),
        out_shape=jax.ShapeDtypeStruct((N_P, D_HID), jnp.float32),
    )(degp, x, W1)

    zp1 = _agg_kernel(y1, er)

    y2 = pl.pallas_call(
        _hidden_body,
        grid=_GRID,
        in_specs=[
            _degp_spec(),
            _zp_spec(D_HID),
            _row_spec(D_HID),
            _full_spec((1, D_HID)),
        ],
        out_specs=_row_spec(D_HID),
        out_shape=jax.ShapeDtypeStruct((N_P, D_HID), jnp.float32),
    )(degp, zp1, y1, b1.reshape(1, D_HID))

    zp2 = _agg_kernel(y2, er)

    out = pl.pallas_call(
        _out_body,
        grid=_GRID,
        in_specs=[
            _degp_spec(),
            _zp_spec(D_HID),
            _row_spec(D_HID),
            _full_spec((D_HID, D_OUT)),
            _full_spec((1, D_OUT)),
        ],
        out_specs=_row_spec(D_OUT),
        out_shape=jax.ShapeDtypeStruct((N_NODES, D_OUT), jnp.float32),
    )(degp, zp2, y2, W2, b2.reshape(1, D_OUT))

    return out
